# baseline (speedup 1.0000x reference)
import functools
import math

import jax
import jax.numpy as jnp
from jax import lax
from jax.experimental import pallas as pl
from jax.experimental.pallas import tpu as pltpu
from jax.experimental.pallas import tpu_sc as plsc

_VOCAB = 100000
_HID = 1024
_BSZ = 4
_QLEN = 2048
_NIDS = _BSZ * _QLEN
_NW = 32
_IDS_PER_W = _NIDS // _NW
_CHUNK = 32
_NCHUNK = _IDS_PER_W // _CHUNK


def _gather_body(ids_hbm, table_hbm, out_hbm, idx_v, buf_a, buf_b, sem_g, sem_s):
    wid = lax.axis_index("s") * 2 + lax.axis_index("c")
    base = wid * _IDS_PER_W
    pltpu.sync_copy(ids_hbm.at[wid], idx_v)

    bufs = [buf_a, buf_b]
    gathers = [None] * _NCHUNK
    stores = [None] * _NCHUNK
    gathers[0] = pltpu.async_copy(table_hbm.at[idx_v.at[0]], bufs[0], sem_g)
    for c in range(_NCHUNK):
        if c + 1 < _NCHUNK:
            if c >= 1:
                stores[c - 1].wait()
            gathers[c + 1] = pltpu.async_copy(
                table_hbm.at[idx_v.at[c + 1]], bufs[(c + 1) % 2], sem_g)
        gathers[c].wait()
        stores[c] = pltpu.async_copy(
            bufs[c % 2], out_hbm.at[pl.ds(base + c * _CHUNK, _CHUNK)], sem_s)
    stores[_NCHUNK - 2].wait()
    stores[_NCHUNK - 1].wait()


@functools.partial(jax.jit, static_argnums=())
def _sc_gather(ids_flat, table):
    mesh = plsc.VectorSubcoreMesh(core_axis_name="c", subcore_axis_name="s")
    run = pl.kernel(
        _gather_body,
        out_type=jax.ShapeDtypeStruct((_NIDS, _HID), jnp.float32),
        mesh=mesh,
        scratch_types=[
            pltpu.VMEM((_NCHUNK, _CHUNK), jnp.int32),
            pltpu.VMEM((_CHUNK, _HID), jnp.float32),
            pltpu.VMEM((_CHUNK, _HID), jnp.float32),
            pltpu.SemaphoreType.DMA,
            pltpu.SemaphoreType.DMA,
        ],
    )
    return run(ids_flat.reshape(_NW, _NCHUNK, _CHUNK), table)


_POS_ROWS = 2 * _QLEN * _BSZ
_POS_BR = 2048


def _pos_body(o_ref):
    i = pl.program_id(0)
    k = lax.broadcasted_iota(jnp.int32, (_POS_BR, _HID), 0) + i * _POS_BR
    c = lax.broadcasted_iota(jnp.int32, (_POS_BR, _HID), 1)
    s = (_QLEN - k // _BSZ).astype(jnp.float32)
    half = _HID // 2
    cm = jnp.where(c < half, c, c - half).astype(jnp.float32)
    inv_freq = jnp.exp(cm * (-math.log(10000.0) / half))
    angle = s * inv_freq
    o_ref[...] = jnp.where(c < half, jnp.sin(angle), jnp.cos(angle))


def _pos_emb():
    out = pl.pallas_call(
        _pos_body,
        out_shape=jax.ShapeDtypeStruct((_POS_ROWS, _HID), jnp.float32),
        grid=(_POS_ROWS // _POS_BR,),
        out_specs=pl.BlockSpec((_POS_BR, _HID), lambda i: (i, 0)),
    )()
    return out.reshape(_BSZ, 2 * _QLEN, _HID)


def kernel(token_id_input, W):
    word = _sc_gather(token_id_input.reshape(-1), W)
    pos = _pos_emb()
    return (word.reshape(_BSZ, _QLEN, _HID), pos)


# speedup vs baseline: 2.8417x; 2.8417x over previous
import functools
import math

import jax
import jax.numpy as jnp
from jax import lax
from jax.experimental import pallas as pl
from jax.experimental.pallas import tpu as pltpu
from jax.experimental.pallas import tpu_sc as plsc

_VOCAB = 100000
_HID = 1024
_BSZ = 4
_QLEN = 2048
_NIDS = _BSZ * _QLEN
_NW = 32
_IDS_PER_W = _NIDS // _NW
_CHUNK = 32
_NCHUNK = _IDS_PER_W // _CHUNK


def _gather_body(ids_hbm, table_hbm, out_hbm, idx_v, buf_a, buf_b, sem_g, sem_s):
    wid = lax.axis_index("s") * 2 + lax.axis_index("c")
    base = wid * _IDS_PER_W
    pltpu.sync_copy(ids_hbm.at[wid], idx_v)

    bufs = [buf_a, buf_b]
    gathers = [None] * _NCHUNK
    stores = [None] * _NCHUNK
    gathers[0] = pltpu.async_copy(table_hbm.at[idx_v.at[0]], bufs[0], sem_g)
    for c in range(_NCHUNK):
        if c + 1 < _NCHUNK:
            if c >= 1:
                stores[c - 1].wait()
            gathers[c + 1] = pltpu.async_copy(
                table_hbm.at[idx_v.at[c + 1]], bufs[(c + 1) % 2], sem_g)
        gathers[c].wait()
        stores[c] = pltpu.async_copy(
            bufs[c % 2], out_hbm.at[pl.ds(base + c * _CHUNK, _CHUNK)], sem_s)
    stores[_NCHUNK - 2].wait()
    stores[_NCHUNK - 1].wait()


@functools.partial(jax.jit, static_argnums=())
def _sc_gather(ids_flat, table):
    mesh = plsc.VectorSubcoreMesh(core_axis_name="c", subcore_axis_name="s")
    run = pl.kernel(
        _gather_body,
        out_type=jax.ShapeDtypeStruct((_NIDS, _HID), jnp.float32),
        mesh=mesh,
        scratch_types=[
            pltpu.VMEM((_NCHUNK, _CHUNK), jnp.int32),
            pltpu.VMEM((_CHUNK, _HID), jnp.float32),
            pltpu.VMEM((_CHUNK, _HID), jnp.float32),
            pltpu.SemaphoreType.DMA,
            pltpu.SemaphoreType.DMA,
        ],
    )
    return run(ids_flat.reshape(_NW, _NCHUNK, _CHUNK), table)


_POS_ROWS = 2 * _QLEN * _BSZ
_HALF = _HID // 2
_UR2 = 256


def _pos_body(o_ref):
    i = pl.program_id(0)
    u = lax.broadcasted_iota(jnp.int32, (_UR2, _HALF), 0) + i * _UR2
    c = lax.broadcasted_iota(jnp.int32, (_UR2, _HALF), 1)
    s_even = (_QLEN - 2 * u).astype(jnp.float32)
    inv_freq = jnp.exp(c.astype(jnp.float32) * (-math.log(10000.0) / _HALF))
    ang_e = s_even * inv_freq
    ang_o = ang_e - inv_freq
    pe_e = jnp.concatenate([jnp.sin(ang_e), jnp.cos(ang_e)], axis=1)
    pe_o = jnp.concatenate([jnp.sin(ang_o), jnp.cos(ang_o)], axis=1)
    sub = lax.broadcasted_iota(jnp.int32, (_UR2, 8, _HID), 1)
    o_ref[...] = jnp.where(sub < 4, pe_e[:, None, :], pe_o[:, None, :])


def _pos_emb():
    out = pl.pallas_call(
        _pos_body,
        out_shape=jax.ShapeDtypeStruct((_POS_ROWS // 8, 8, _HID), jnp.float32),
        grid=(_POS_ROWS // (8 * _UR2),),
        out_specs=pl.BlockSpec((_UR2, 8, _HID), lambda i: (i, 0, 0)),
    )()
    return out.reshape(_BSZ, 2 * _QLEN, _HID)


def kernel(token_id_input, W):
    word = _sc_gather(token_id_input.reshape(-1), W)
    pos = _pos_emb()
    return (word.reshape(_BSZ, _QLEN, _HID), pos)
